# Initial kernel scaffold; baseline (speedup 1.0000x reference)
import jax
import jax.numpy as jnp
from jax.experimental import pallas as pl


def kernel(inputs_0, inputs_1, inputs_2, inputs_3, inputs_4, embed_table, dense_W, dense_b):
    raise NotImplementedError("write your pallas kernel here")


# speedup vs baseline: 99.3906x; 99.3906x over previous
import functools

import jax
import jax.numpy as jnp
from jax import lax
from jax.experimental import pallas as pl
from jax.experimental.pallas import tpu as pltpu
from jax.experimental.pallas import tpu_sc as plsc

N_ITEMS = 1000
DIM = 10
N_IN = 5
BATCH = 16384
HIST = 50
P = N_IN * HIST

NC = 2
NS = 16
L = 16
NW = NC * NS
BPW = BATCH // NW
HP = 56
PP = N_IN * HP


def _s_body(wr_ref, et_ref, b_ref, s_ref):
    s_ref[...] = (
        jnp.dot(wr_ref[...], et_ref[...], preferred_element_type=jnp.float32)
        + b_ref[0, 0] * (1.0 / P)
    )


def _compute_s(wr, et, bias):
    return pl.pallas_call(
        _s_body,
        out_shape=jax.ShapeDtypeStruct((PP, N_ITEMS), jnp.float32),
    )(wr, et, bias)


@functools.partial(
    pl.kernel,
    out_type=jax.ShapeDtypeStruct((BATCH,), jnp.float32),
    mesh=plsc.VectorSubcoreMesh(core_axis_name="c", subcore_axis_name="s"),
    compiler_params=pltpu.CompilerParams(needs_layout_passes=False),
    scratch_types=[
        pltpu.VMEM((HP, N_ITEMS), jnp.float32),
        pltpu.VMEM((BPW, HIST), jnp.int32),
        pltpu.VMEM((BPW,), jnp.float32),
    ],
)
def _sc_gather_sum(i0, i1, i2, i3, i4, s_hbm, out_hbm, sbuf, ibuf, acc):
    wid = lax.axis_index("s") * NC + lax.axis_index("c")
    base = wid * BPW
    iota = lax.iota(jnp.int32, L)
    idx_refs = [i0, i1, i2, i3, i4]
    for i in range(N_IN):
        pltpu.sync_copy(s_hbm.at[pl.ds(i * HP, HP)], sbuf)
        pltpu.sync_copy(idx_refs[i].at[pl.ds(base, BPW)], ibuf)

        def g_body(g, _, first=(i == 0)):
            bvec = g * L + iota
            accv = jnp.zeros((L,), jnp.float32)
            for h in range(HIST):
                hv = jnp.full((L,), h, jnp.int32)
                iv = plsc.load_gather(ibuf, [bvec, hv])
                accv = accv + plsc.load_gather(sbuf, [hv, iv])
            sl = pl.ds(g * L, L)
            if first:
                acc[sl] = accv
            else:
                acc[sl] = acc[sl] + accv
            return _

        lax.fori_loop(0, BPW // L, g_body, None)
    pltpu.sync_copy(acc, out_hbm.at[pl.ds(base, BPW)])


def kernel(inputs_0, inputs_1, inputs_2, inputs_3, inputs_4,
           embed_table, dense_W, dense_b):
    wr = dense_W.reshape(N_IN, HIST, DIM)
    wr = jnp.pad(wr, ((0, 0), (0, HP - HIST), (0, 0))).reshape(PP, DIM)
    et = embed_table.T
    bias = dense_b.reshape(1, 1)
    s = _compute_s(wr, et, bias)
    out = _sc_gather_sum(inputs_0, inputs_1, inputs_2, inputs_3, inputs_4, s)
    return out.reshape(BATCH, 1)


# speedup vs baseline: 102.0652x; 1.0269x over previous
import functools

import jax
import jax.numpy as jnp
from jax import lax
from jax.experimental import pallas as pl
from jax.experimental.pallas import tpu as pltpu
from jax.experimental.pallas import tpu_sc as plsc

N_ITEMS = 1000
DIM = 10
N_IN = 5
BATCH = 16384
HIST = 50
P = N_IN * HIST

NC = 2
NS = 16
L = 16
NW = NC * NS
BPW = BATCH // NW
GPW = BPW // L
SROWS = HIST * N_ITEMS


def _s_body(wr_ref, et_ref, b_ref, s_ref):
    s_ref[...] = (
        jnp.dot(wr_ref[...], et_ref[...], preferred_element_type=jnp.float32)
        + b_ref[0, 0] * (1.0 / P)
    )


def _compute_s(wr, et, bias):
    return pl.pallas_call(
        _s_body,
        out_shape=jax.ShapeDtypeStruct((P, N_ITEMS), jnp.float32),
    )(wr, et, bias)


@functools.partial(
    pl.kernel,
    out_type=jax.ShapeDtypeStruct((BATCH,), jnp.float32),
    mesh=plsc.VectorSubcoreMesh(core_axis_name="c", subcore_axis_name="s"),
    compiler_params=pltpu.CompilerParams(needs_layout_passes=False),
    scratch_types=[
        pltpu.VMEM((SROWS,), jnp.float32),
        pltpu.VMEM((BPW * HIST,), jnp.int32),
        pltpu.VMEM((BPW,), jnp.float32),
    ],
)
def _sc_gather_sum(i0, i1, i2, i3, i4, s_hbm, out_hbm, sbuf, ibuf, acc):
    wid = lax.axis_index("s") * NC + lax.axis_index("c")
    base = wid * BPW
    lane50 = lax.iota(jnp.int32, L) * HIST
    idx_refs = [i0, i1, i2, i3, i4]
    for i in range(N_IN):
        pltpu.sync_copy(s_hbm.at[pl.ds(i * SROWS, SROWS)], sbuf)
        pltpu.sync_copy(idx_refs[i].at[pl.ds(base * HIST, BPW * HIST)], ibuf)

        def g_body(g, _, first=(i == 0)):
            addr_g = g * (L * HIST) + lane50
            accv = jnp.zeros((L,), jnp.float32)
            for h in range(HIST):
                iv = plsc.load_gather(ibuf, [addr_g + h])
                accv = accv + plsc.load_gather(sbuf, [iv + h * N_ITEMS])
            sl = pl.ds(g * L, L)
            if first:
                acc[sl] = accv
            else:
                acc[sl] = acc[sl] + accv
            return _

        lax.fori_loop(0, GPW, g_body, None)
    pltpu.sync_copy(acc, out_hbm.at[pl.ds(base, BPW)])


def kernel(inputs_0, inputs_1, inputs_2, inputs_3, inputs_4,
           embed_table, dense_W, dense_b):
    wr = dense_W.reshape(P, DIM)
    et = embed_table.T
    bias = dense_b.reshape(1, 1)
    s = _compute_s(wr, et, bias).reshape(P * N_ITEMS)
    out = _sc_gather_sum(
        inputs_0.reshape(-1), inputs_1.reshape(-1), inputs_2.reshape(-1),
        inputs_3.reshape(-1), inputs_4.reshape(-1), s)
    return out.reshape(BATCH, 1)


# speedup vs baseline: 109.3530x; 1.0714x over previous
import functools

import jax
import jax.numpy as jnp
from jax import lax
from jax.experimental import pallas as pl
from jax.experimental.pallas import tpu as pltpu
from jax.experimental.pallas import tpu_sc as plsc

N_ITEMS = 1000
DIM = 10
N_IN = 5
BATCH = 16384
HIST = 50
P = N_IN * HIST

NC = 2
NS = 16
L = 16
NW = NC * NS
BPW = BATCH // NW
SROWS = HIST * N_ITEMS
CH = 256
NCH = BPW // CH


def _s_body(wr_ref, et_ref, b_ref, s_ref):
    s_ref[...] = (
        jnp.dot(wr_ref[...], et_ref[...], preferred_element_type=jnp.float32)
        + b_ref[0, 0] * (1.0 / P)
    )


def _compute_s(wr, et, bias):
    return pl.pallas_call(
        _s_body,
        out_shape=jax.ShapeDtypeStruct((P, N_ITEMS), jnp.float32),
    )(wr, et, bias)


@functools.partial(
    pl.kernel,
    out_type=jax.ShapeDtypeStruct((BATCH,), jnp.float32),
    mesh=plsc.VectorSubcoreMesh(core_axis_name="c", subcore_axis_name="s"),
    compiler_params=pltpu.CompilerParams(needs_layout_passes=False),
    scratch_types=[
        pltpu.VMEM((SROWS,), jnp.float32),
        pltpu.VMEM((CH, HIST), jnp.int32),
        pltpu.VMEM((CH, HIST), jnp.int32),
        pltpu.VMEM((BPW,), jnp.float32),
        pltpu.SemaphoreType.DMA,
        pltpu.SemaphoreType.DMA,
    ],
)
def _sc_gather_sum(i0, i1, i2, i3, i4, s_hbm, out_hbm,
                   sbuf, ib0, ib1, acc, sem0, sem1):
    wid = lax.axis_index("s") * NC + lax.axis_index("c")
    base = wid * BPW
    iota = lax.iota(jnp.int32, L)
    idx_refs = [i0, i1, i2, i3, i4]
    ibufs = [ib0, ib1]
    sems = [sem0, sem1]
    steps = [(i, cb) for i in range(N_IN) for cb in range(NCH)]

    def start(k):
        i, cb = steps[k]
        return pltpu.async_copy(
            idx_refs[i].at[pl.ds(base + cb * CH, CH)],
            ibufs[k % 2], sems[k % 2])

    handle = start(0)
    for k, (i, cb) in enumerate(steps):
        if cb == 0:
            pltpu.sync_copy(s_hbm.at[pl.ds(i * SROWS, SROWS)], sbuf)
        nxt = start(k + 1) if k + 1 < len(steps) else None
        handle.wait()
        ibuf = ibufs[k % 2]

        def g_body(g, _, ibuf=ibuf, first=(i == 0), cb=cb):
            bvec = g * L + iota
            accv = jnp.zeros((L,), jnp.float32)
            for h in range(HIST):
                iv = plsc.load_gather(ibuf, [bvec, jnp.full((L,), h, jnp.int32)])
                accv = accv + plsc.load_gather(sbuf, [iv + h * N_ITEMS])
            sl = pl.ds(cb * CH + g * L, L)
            if first:
                acc[sl] = accv
            else:
                acc[sl] = acc[sl] + accv
            return _

        lax.fori_loop(0, CH // L, g_body, None)
        handle = nxt
    pltpu.sync_copy(acc, out_hbm.at[pl.ds(base, BPW)])


def kernel(inputs_0, inputs_1, inputs_2, inputs_3, inputs_4,
           embed_table, dense_W, dense_b):
    wr = dense_W.reshape(P, DIM)
    et = embed_table.T
    bias = dense_b.reshape(1, 1)
    s = _compute_s(wr, et, bias).reshape(P * N_ITEMS)
    out = _sc_gather_sum(inputs_0, inputs_1, inputs_2, inputs_3, inputs_4, s)
    return out.reshape(BATCH, 1)


# speedup vs baseline: 239.5807x; 2.1909x over previous
import functools

import jax
import jax.numpy as jnp
from jax import lax
from jax.experimental import pallas as pl
from jax.experimental.pallas import tpu as pltpu
from jax.experimental.pallas import tpu_sc as plsc

N_ITEMS = 1000
DIM = 10
N_IN = 5
BATCH = 16384
HIST = 50
P = N_IN * HIST

NC = 2
NS = 16
L = 16
NW = NC * NS
BPW = BATCH // NW
SROWS = HIST * N_ITEMS


def _s_body(wr_ref, et_ref, b_ref, s_ref):
    s_ref[...] = (
        jnp.dot(wr_ref[...], et_ref[...], preferred_element_type=jnp.float32)
        + b_ref[0, 0] * (1.0 / P)
    )


def _compute_s(wr, et, bias):
    return pl.pallas_call(
        _s_body,
        out_shape=jax.ShapeDtypeStruct((P, N_ITEMS), jnp.float32),
    )(wr, et, bias)


@functools.partial(
    pl.kernel,
    out_type=jax.ShapeDtypeStruct((BATCH,), jnp.float32),
    mesh=plsc.VectorSubcoreMesh(core_axis_name="c", subcore_axis_name="s"),
    compiler_params=pltpu.CompilerParams(needs_layout_passes=False),
    scratch_types=[
        pltpu.VMEM((SROWS,), jnp.float32),
        pltpu.VMEM((HIST, BPW), jnp.int32),
        pltpu.VMEM((HIST, BPW), jnp.int32),
        pltpu.VMEM((BPW,), jnp.float32),
        pltpu.SemaphoreType.DMA,
        pltpu.SemaphoreType.DMA,
    ],
)
def _sc_gather_sum(i0, i1, i2, i3, i4, s_hbm, out_hbm,
                   sbuf, ib0, ib1, acc, sem0, sem1):
    wid = lax.axis_index("s") * NC + lax.axis_index("c")
    base = wid * BPW
    iota = lax.iota(jnp.int32, L)
    idx_refs = [i0, i1, i2, i3, i4]
    ibufs = [ib0, ib1]
    sems = [sem0, sem1]

    def start(i):
        return pltpu.async_copy(
            idx_refs[i].at[:, pl.ds(base, BPW)], ibufs[i % 2], sems[i % 2])

    handle = start(0)
    for i in range(N_IN):
        pltpu.sync_copy(s_hbm.at[pl.ds(i * SROWS, SROWS)], sbuf)
        nxt = start(i + 1) if i + 1 < N_IN else None
        handle.wait()
        ibuf = ibufs[i % 2]

        def g_body(g, _, ibuf=ibuf, first=(i == 0)):
            bvec = g * L + iota
            accv = jnp.zeros((L,), jnp.float32)
            for h in range(HIST):
                iv = plsc.load_gather(ibuf, [jnp.full((L,), h, jnp.int32), bvec])
                accv = accv + plsc.load_gather(sbuf, [iv + h * N_ITEMS])
            sl = pl.ds(g * L, L)
            if first:
                acc[sl] = accv
            else:
                acc[sl] = acc[sl] + accv
            return _

        lax.fori_loop(0, BPW // L, g_body, None)
        handle = nxt
    pltpu.sync_copy(acc, out_hbm.at[pl.ds(base, BPW)])


def kernel(inputs_0, inputs_1, inputs_2, inputs_3, inputs_4,
           embed_table, dense_W, dense_b):
    wr = dense_W.reshape(P, DIM)
    et = embed_table.T
    bias = dense_b.reshape(1, 1)
    s = _compute_s(wr, et, bias).reshape(P * N_ITEMS)
    out = _sc_gather_sum(inputs_0.T, inputs_1.T, inputs_2.T,
                         inputs_3.T, inputs_4.T, s)
    return out.reshape(BATCH, 1)
